# baseline (speedup 1.0000x reference)
import functools

import jax
import jax.numpy as jnp
from jax import lax
from jax.experimental import pallas as pl
from jax.experimental.pallas import tpu as pltpu
from jax.experimental.pallas import tpu_sc as plsc

D = 32
BATCH = 16384
HIST = 50
NC = 2
NS = 16
NW = NC * NS
BPW = BATCH // NW
CB = 4
NBUF = 2
LPAD = 129
SCALE = float(D) ** 0.5


@functools.partial(
    pl.kernel,
    mesh=plsc.VectorSubcoreMesh(core_axis_name="c", subcore_axis_name="s"),
    out_type=jax.ShapeDtypeStruct((HIST, D // 8, BATCH // 128, 8, 128),
                                  jnp.float32),
    compiler_params=pltpu.CompilerParams(
        use_tc_tiling_on_sc=False, needs_layout_passes=False
    ),
    scratch_types=[
        pltpu.VMEM((HIST, CB, 128), jnp.int32),
        pltpu.VMEM((NBUF, CB, 128, D), jnp.float32),
        pltpu.VMEM((NBUF, D // 8, CB, 8, LPAD), jnp.float32),
    ]
    + [pltpu.SemaphoreType.DMA] * (2 * NBUF),
)
def _emb_lookup(idx_hbm, table_hbm, out_hbm, idx_v, rin, obuf, *sems):
    gsems = sems[:NBUF]
    osems = sems[NBUF:]
    wid = lax.axis_index("s") * NC + lax.axis_index("c")
    c0 = wid * CB
    iota = lax.iota(jnp.int32, 16)

    for c in range(CB):
        pltpu.sync_copy(
            idx_hbm.at[:, pl.ds((c0 + c) * 128, 128)], idx_v.at[:, c]
        )

    def gstart(h, slot):
        for c in range(CB):
            pltpu.async_copy(
                table_hbm.at[idx_v.at[h, c]], rin.at[slot, c], gsems[slot]
            )

    def gwait(slot):
        for c in range(CB):
            pltpu.make_async_copy(
                table_hbm.at[pl.ds(0, 128)], rin.at[slot, c], gsems[slot]
            ).wait()

    def ostart(h, slot):
        pltpu.async_copy(
            obuf.at[slot, :, :, :, pl.ds(0, 128)],
            out_hbm.at[h, :, pl.ds(c0, CB)],
            osems[slot],
        )

    def owait(slot):
        pltpu.make_async_copy(
            obuf.at[slot, :, :, :, pl.ds(0, 128)],
            out_hbm.at[0, :, pl.ds(c0, CB)],
            osems[slot],
        ).wait()

    bands = [jax.lax.shift_right_logical(iota, 3) + 2 * t for t in range(2)]
    subl = jax.lax.bitwise_and(iota, jnp.full((16,), 7, jnp.int32))
    cvs = [jnp.full((16,), c, jnp.int32) for c in range(CB)]

    def transpose_scale(slot):
        def row_body(l, _):
            lv = jnp.full((16,), l, jnp.int32)
            for c in range(CB):
                for t in range(2):
                    v = rin[slot, c, l, pl.ds(16 * t, 16)] * SCALE
                    plsc.store_scatter(
                        obuf.at[slot], [bands[t], cvs[c], subl, lv], v
                    )
            return ()

        lax.fori_loop(0, 128, row_body, ())

    for slot in range(NBUF):
        gstart(slot, slot)
    for slot in range(NBUF):
        gwait(slot)
        transpose_scale(slot)
        ostart(slot, slot)
        gstart(slot + NBUF, slot)

    def pair(g, _):
        for slot in range(NBUF):
            h = NBUF * g + slot
            gwait(slot)
            owait(slot)
            transpose_scale(slot)
            ostart(h, slot)
            gstart(h + NBUF, slot)
        return ()

    lax.fori_loop(1, HIST // NBUF - 1, pair, ())

    for slot in range(NBUF):
        h = HIST - NBUF + slot
        gwait(slot)
        owait(slot)
        transpose_scale(slot)
        ostart(h, slot)
    for slot in range(NBUF):
        owait(slot)


def kernel(inputs, embeddings):
    idx = inputs.astype(jnp.int32).T
    out5 = _emb_lookup(idx, embeddings)
    return out5.transpose(2, 4, 0, 1, 3).reshape(BATCH, HIST, D)
